# baseline (speedup 1.0000x reference)
import functools

import jax
import jax.numpy as jnp
from jax import lax
from jax.experimental import pallas as pl
from jax.experimental.pallas import tpu as pltpu
from jax.experimental.pallas import tpu_sc as plsc

NC, NS, LANES = 2, 16, 16
NW = NC * NS
IROW = 128
SUB = 4
CHUNK = IROW * SUB


def _make_transpose_kernel(v: int, d: int):
    nblk = v // 128
    per_w = nblk // NW
    extra = nblk - per_w * NW
    tail_rows = v - nblk * 128
    wide_rows = v * d // 128
    out_sds = jax.ShapeDtypeStruct((wide_rows, 128), jnp.float32)

    @functools.partial(
        pl.kernel,
        out_type=(out_sds, out_sds),
        mesh=plsc.VectorSubcoreMesh(
            core_axis_name="c", subcore_axis_name="s",
            num_cores=NC, num_subcores=NS),
        scratch_types=[
            pltpu.VMEM((d, 128), jnp.float32),
            pltpu.VMEM((d, 128), jnp.float32),
            pltpu.VMEM((d, 128), jnp.float32),
            pltpu.VMEM((d, 128), jnp.float32),
            pltpu.VMEM((d * 128 // 128, 128), jnp.float32),
            pltpu.VMEM((d * 128 // 128, 128), jnp.float32),
            pltpu.VMEM((d * 128 // 128, 128), jnp.float32),
            pltpu.VMEM((d * 128 // 128, 128), jnp.float32),
            pltpu.VMEM((16, 128), jnp.float32),
            pltpu.SemaphoreType.DMA,
            pltpu.SemaphoreType.DMA,
            pltpu.SemaphoreType.DMA,
            pltpu.SemaphoreType.DMA,
        ],
        compiler_params=pltpu.CompilerParams(
            use_tc_tiling_on_sc=True, needs_layout_passes=False),
    )
    def k(ett_hbm, etail_hbm, gtt_hbm, gtail_hbm, eout_hbm, gout_hbm,
          ein0, ein1, gin0, gin1, eo0, eo1, go0, go1, tbuf,
          gs0, gs1, os0, os1):
        wid = lax.axis_index("s") * NC + lax.axis_index("c")
        base = wid * per_w
        inb = ((ein0, gin0), (ein1, gin1))
        outb = ((eo0, go0), (eo1, go1))
        gsem = (gs0, gs1)
        osem = (os0, os1)
        srcs = (ett_hbm, gtt_hbm)
        dsts = (eout_hbm, gout_hbm)
        obr = d * 128 // 128

        def fire_in(blk, slot):
            for t in range(2):
                pltpu.async_copy(
                    srcs[t].at[:, pl.ds(blk * 128, 128)], inb[slot][t],
                    gsem[slot])

        def wait_in(slot):
            for t in range(2):
                pltpu.make_async_copy(
                    srcs[t].at[:, pl.ds(0, 128)], inb[slot][t],
                    gsem[slot]).wait()

        def transpose(slot):
            def row(R, carry):
                for t in range(2):
                    for j in range(d // 4):
                        cidx = lax.iota(jnp.int32, LANES) + (j % 2) * LANES
                        ridx = jnp.full((LANES,), 4 * R + j // 2, jnp.int32)
                        outb[slot][t][R, pl.ds(j * LANES, LANES)] = (
                            plsc.load_gather(inb[slot][t], [cidx, ridx]))
                return carry

            lax.fori_loop(0, obr, row, 0, unroll=2)

        def fire_out(blk, slot):
            for t in range(2):
                pltpu.async_copy(
                    outb[slot][t], dsts[t].at[pl.ds(blk * obr, obr)],
                    osem[slot])

        def wait_out(slot):
            for t in range(2):
                pltpu.make_async_copy(
                    outb[slot][t], dsts[t].at[pl.ds(0, obr)],
                    osem[slot]).wait()

        @pl.when(wid < extra)
        def _():
            blk = per_w * NW + wid
            fire_in(blk, 0)
            wait_in(0)
            transpose(0)
            fire_out(blk, 0)
            wait_out(0)

        if tail_rows:
            trow = tail_rows * d // 128
            for t in range(2):
                tails = (etail_hbm, gtail_hbm)

                @pl.when(wid == extra + t)
                def _(t=t):
                    pltpu.sync_copy(tails[t], tbuf.at[pl.ds(0, trow)])
                    pltpu.sync_copy(tbuf.at[pl.ds(0, trow)],
                                    dsts[t].at[pl.ds(nblk * obr, trow)])

        fire_in(base, 0)

        def loop_body(j, carry):
            for s in range(2):
                blk = base + j + s
                nxt = j + s + 1
                nslot = 1 - s

                @pl.when(nxt < per_w)
                def _():
                    @pl.when(j + s >= 1)
                    def _():
                        wait_out(nslot)
                    fire_in(base + nxt, nslot)

                wait_in(s)
                transpose(s)
                fire_out(blk, s)
            return carry

        lax.fori_loop(0, per_w // 2, lambda i, c: loop_body(2 * i, c), 0)
        wait_out(0)
        wait_out(1)

    return k


def _make_gather_kernel(n_chunks: int, d: int):
    per_w = n_chunks * CHUNK

    @functools.partial(
        pl.kernel,
        out_type=jax.ShapeDtypeStruct((NW * per_w, d), jnp.float32),
        mesh=plsc.VectorSubcoreMesh(
            core_axis_name="c", subcore_axis_name="s",
            num_cores=NC, num_subcores=NS),
        scratch_types=[
            pltpu.VMEM((n_chunks * SUB, IROW), jnp.int32),
            pltpu.VMEM((CHUNK, d), jnp.float32),
            pltpu.VMEM((CHUNK, d), jnp.float32),
            pltpu.VMEM((CHUNK, d), jnp.float32),
            pltpu.VMEM((CHUNK, d), jnp.float32),
            pltpu.SemaphoreType.DMA,
            pltpu.SemaphoreType.DMA,
            pltpu.SemaphoreType.DMA,
            pltpu.SemaphoreType.DMA,
        ],
        compiler_params=pltpu.CompilerParams(use_tc_tiling_on_sc=False),
    )
    def k(idx_hbm, emb_hbm, gate_hbm, out_hbm,
          idx_v, emb0, emb1, gate0, gate1, gs0, gs1, os0, os1):
        wid = lax.axis_index("s") * NC + lax.axis_index("c")
        base = wid * per_w
        ebuf = (emb0, emb1)
        gbuf = (gate0, gate1)
        gsem = (gs0, gs1)
        osem = (os0, os1)

        pltpu.sync_copy(idx_hbm.at[wid], idx_v)

        def fire_gathers(chunk, slot):
            for q in range(SUB):
                row = idx_v.at[chunk * SUB + q]
                dst = pl.ds(q * IROW, IROW)
                pltpu.async_copy(emb_hbm.at[row], ebuf[slot].at[dst], gsem[slot])
                pltpu.async_copy(gate_hbm.at[row], gbuf[slot].at[dst], gsem[slot])

        def wait_gathers(slot):
            row = idx_v.at[0]
            pltpu.make_async_copy(emb_hbm.at[row], ebuf[slot], gsem[slot]).wait()
            pltpu.make_async_copy(gate_hbm.at[row], gbuf[slot], gsem[slot]).wait()

        def out_slice(chunk):
            return out_hbm.at[pl.ds(base + chunk * CHUNK, CHUNK)]

        def compute_chunk(slot):
            e, g = ebuf[slot], gbuf[slot]

            def row_body(r, carry):
                for h in range(0, d, LANES):
                    sl = (r, pl.ds(h, LANES))
                    e[sl] = jnp.where(g[sl] >= 0.5, e[sl], 0.0)
                return carry

            lax.fori_loop(0, CHUNK, row_body, 0, unroll=8)

        fire_gathers(0, 0)

        def loop_body(j, carry):
            for b in range(2):
                chunk = j + b
                nxt = chunk + 1
                nslot = 1 - b

                @pl.when(nxt < n_chunks)
                def _():
                    @pl.when(chunk >= 1)
                    def _():
                        pltpu.make_async_copy(
                            ebuf[nslot], out_slice(chunk - 1), osem[nslot]
                        ).wait()
                    fire_gathers(nxt, nslot)

                wait_gathers(b)
                compute_chunk(b)
                pltpu.async_copy(ebuf[b], out_slice(chunk), osem[b])
            return carry

        lax.fori_loop(0, n_chunks // 2, lambda i, c: loop_body(2 * i, c), 0)

        pltpu.make_async_copy(ebuf[0], out_slice(n_chunks - 2), osem[0]).wait()
        pltpu.make_async_copy(ebuf[1], out_slice(n_chunks - 1), osem[1]).wait()

    return k


def kernel(indices, emb_table, gate_table):
    b, f = indices.shape
    v, d = emb_table.shape
    n = b * f
    assert n % (NW * CHUNK) == 0 and d % LANES == 0
    assert (v % 128) * d % 128 == 0 and v * d % 128 == 0
    n_chunks = n // (NW * CHUNK)
    nblk = v // 128
    tail_rows = v - nblk * 128

    def tail(t):
        if tail_rows:
            return t[v - tail_rows:, :].reshape(tail_rows * d // 128, 128)
        return jnp.zeros((0, 128), jnp.float32)

    wide_e, wide_g = _make_transpose_kernel(v, d)(
        emb_table.T, tail(emb_table), gate_table.T, tail(gate_table))
    idx = indices.astype(jnp.int32).reshape(NW, n_chunks * SUB, IROW)
    out = _make_gather_kernel(n_chunks, d)(
        idx, wide_e.reshape(v, d), wide_g.reshape(v, d))
    return out.reshape(b, f, d)


# speedup vs baseline: 1.1653x; 1.1653x over previous
import functools

import jax
import jax.numpy as jnp
from jax import lax
from jax.experimental import pallas as pl
from jax.experimental.pallas import tpu as pltpu
from jax.experimental.pallas import tpu_sc as plsc

NC, NS, LANES = 2, 16, 16
NW = NC * NS
IROW = 128
SUB = 4
CHUNK = IROW * SUB


def _make_transpose_kernel(v: int, d: int):
    nblk = v // 128
    per_w = nblk // NW
    extra = nblk - per_w * NW
    tail_rows = v - nblk * 128
    out_sds = jax.ShapeDtypeStruct((v * d,), jnp.float32)

    @functools.partial(
        pl.kernel,
        out_type=(out_sds, out_sds),
        mesh=plsc.VectorSubcoreMesh(
            core_axis_name="c", subcore_axis_name="s",
            num_cores=NC, num_subcores=NS),
        scratch_types=[
            pltpu.VMEM((d, 128), jnp.float32),
            pltpu.VMEM((d, 128), jnp.float32),
            pltpu.VMEM((d, 128), jnp.float32),
            pltpu.VMEM((d, 128), jnp.float32),
            pltpu.VMEM((d * 128,), jnp.float32),
            pltpu.VMEM((d * 128,), jnp.float32),
            pltpu.VMEM((d * 128,), jnp.float32),
            pltpu.VMEM((d * 128,), jnp.float32),
            pltpu.VMEM((tail_rows * d or 128,), jnp.float32),
            pltpu.SemaphoreType.DMA,
            pltpu.SemaphoreType.DMA,
            pltpu.SemaphoreType.DMA,
            pltpu.SemaphoreType.DMA,
        ],
        compiler_params=pltpu.CompilerParams(
            use_tc_tiling_on_sc=True, needs_layout_passes=False),
    )
    def k(ett_hbm, etail_hbm, gtt_hbm, gtail_hbm, eout_hbm, gout_hbm,
          ein0, ein1, gin0, gin1, eo0, eo1, go0, go1, tbuf,
          gs0, gs1, os0, os1):
        wid = lax.axis_index("s") * NC + lax.axis_index("c")
        base = wid * per_w
        inb = ((ein0, gin0), (ein1, gin1))
        outb = ((eo0, go0), (eo1, go1))
        gsem = (gs0, gs1)
        osem = (os0, os1)
        srcs = (ett_hbm, gtt_hbm)
        dsts = (eout_hbm, gout_hbm)
        owords = d * 128
        scat = lax.iota(jnp.int32, LANES) * d

        def fire_in(blk, slot):
            for t in range(2):
                pltpu.async_copy(
                    srcs[t].at[:, pl.ds(blk * 128, 128)], inb[slot][t],
                    gsem[slot])

        def wait_in(slot):
            for t in range(2):
                pltpu.make_async_copy(
                    srcs[t].at[:, pl.ds(0, 128)], inb[slot][t],
                    gsem[slot]).wait()

        def transpose(slot):
            def col(c, carry):
                for t in range(2):
                    for h in range(0, 128, LANES):
                        val = inb[slot][t][c, pl.ds(h, LANES)]
                        plsc.store_scatter(outb[slot][t],
                                           [scat + (h * d + c)], val)
                return carry

            lax.fori_loop(0, d, col, 0, unroll=4)

        def fire_out(blk, slot):
            for t in range(2):
                pltpu.async_copy(
                    outb[slot][t], dsts[t].at[pl.ds(blk * owords, owords)],
                    osem[slot])

        def wait_out(slot):
            for t in range(2):
                pltpu.make_async_copy(
                    outb[slot][t], dsts[t].at[pl.ds(0, owords)],
                    osem[slot]).wait()

        @pl.when(wid < extra)
        def _():
            blk = per_w * NW + wid
            fire_in(blk, 0)
            wait_in(0)
            transpose(0)
            fire_out(blk, 0)
            wait_out(0)

        if tail_rows:
            tw = tail_rows * d
            for t in range(2):
                tails = (etail_hbm, gtail_hbm)

                @pl.when(wid == extra + t)
                def _(t=t):
                    pltpu.sync_copy(tails[t], tbuf)
                    pltpu.sync_copy(tbuf,
                                    dsts[t].at[pl.ds(nblk * owords, tw)])

        fire_in(base, 0)

        def loop_body(j, carry):
            for s in range(2):
                blk = base + j + s
                nxt = j + s + 1
                nslot = 1 - s

                @pl.when(nxt < per_w)
                def _():
                    @pl.when(j + s >= 1)
                    def _():
                        wait_out(nslot)
                    fire_in(base + nxt, nslot)

                wait_in(s)
                transpose(s)
                fire_out(blk, s)
            return carry

        lax.fori_loop(0, per_w // 2, lambda i, c: loop_body(2 * i, c), 0)
        wait_out(0)
        wait_out(1)

    return k


def _make_gather_kernel(n_chunks: int, d: int):
    per_w = n_chunks * CHUNK

    @functools.partial(
        pl.kernel,
        out_type=jax.ShapeDtypeStruct((NW * per_w, d), jnp.float32),
        mesh=plsc.VectorSubcoreMesh(
            core_axis_name="c", subcore_axis_name="s",
            num_cores=NC, num_subcores=NS),
        scratch_types=[
            pltpu.VMEM((n_chunks * SUB, IROW), jnp.int32),
            pltpu.VMEM((CHUNK, d), jnp.float32),
            pltpu.VMEM((CHUNK, d), jnp.float32),
            pltpu.VMEM((CHUNK, d), jnp.float32),
            pltpu.VMEM((CHUNK, d), jnp.float32),
            pltpu.SemaphoreType.DMA,
            pltpu.SemaphoreType.DMA,
            pltpu.SemaphoreType.DMA,
            pltpu.SemaphoreType.DMA,
        ],
        compiler_params=pltpu.CompilerParams(use_tc_tiling_on_sc=False),
    )
    def k(idx_hbm, emb_hbm, gate_hbm, out_hbm,
          idx_v, emb0, emb1, gate0, gate1, gs0, gs1, os0, os1):
        wid = lax.axis_index("s") * NC + lax.axis_index("c")
        base = wid * per_w
        ebuf = (emb0, emb1)
        gbuf = (gate0, gate1)
        gsem = (gs0, gs1)
        osem = (os0, os1)

        pltpu.sync_copy(idx_hbm.at[wid], idx_v)

        def fire_gathers(chunk, slot):
            for q in range(SUB):
                row = idx_v.at[chunk * SUB + q]
                dst = pl.ds(q * IROW, IROW)
                pltpu.async_copy(emb_hbm.at[row], ebuf[slot].at[dst], gsem[slot])
                pltpu.async_copy(gate_hbm.at[row], gbuf[slot].at[dst], gsem[slot])

        def wait_gathers(slot):
            row = idx_v.at[0]
            pltpu.make_async_copy(emb_hbm.at[row], ebuf[slot], gsem[slot]).wait()
            pltpu.make_async_copy(gate_hbm.at[row], gbuf[slot], gsem[slot]).wait()

        def out_slice(chunk):
            return out_hbm.at[pl.ds(base + chunk * CHUNK, CHUNK)]

        def compute_chunk(slot):
            e, g = ebuf[slot], gbuf[slot]

            def row_body(r, carry):
                for h in range(0, d, LANES):
                    sl = (r, pl.ds(h, LANES))
                    e[sl] = jnp.where(g[sl] >= 0.5, e[sl], 0.0)
                return carry

            lax.fori_loop(0, CHUNK, row_body, 0, unroll=8)

        fire_gathers(0, 0)

        def loop_body(j, carry):
            for b in range(2):
                chunk = j + b
                nxt = chunk + 1
                nslot = 1 - b

                @pl.when(nxt < n_chunks)
                def _():
                    @pl.when(chunk >= 1)
                    def _():
                        pltpu.make_async_copy(
                            ebuf[nslot], out_slice(chunk - 1), osem[nslot]
                        ).wait()
                    fire_gathers(nxt, nslot)

                wait_gathers(b)
                compute_chunk(b)
                pltpu.async_copy(ebuf[b], out_slice(chunk), osem[b])
            return carry

        lax.fori_loop(0, n_chunks // 2, lambda i, c: loop_body(2 * i, c), 0)

        pltpu.make_async_copy(ebuf[0], out_slice(n_chunks - 2), osem[0]).wait()
        pltpu.make_async_copy(ebuf[1], out_slice(n_chunks - 1), osem[1]).wait()

    return k


def kernel(indices, emb_table, gate_table):
    b, f = indices.shape
    v, d = emb_table.shape
    n = b * f
    assert n % (NW * CHUNK) == 0 and d % LANES == 0
    assert (v % 128) * d % 128 == 0 and v * d % 128 == 0
    n_chunks = n // (NW * CHUNK)
    nblk = v // 128
    tail_rows = v - nblk * 128

    def tail(t):
        if tail_rows:
            return t[v - tail_rows:, :].reshape(tail_rows * d)
        return jnp.zeros((128,), jnp.float32)

    wide_e, wide_g = _make_transpose_kernel(v, d)(
        emb_table.T, tail(emb_table), gate_table.T, tail(gate_table))
    idx = indices.astype(jnp.int32).reshape(NW, n_chunks * SUB, IROW)
    out = _make_gather_kernel(n_chunks, d)(
        idx, wide_e.reshape(v, d), wide_g.reshape(v, d))
    return out.reshape(b, f, d)


# speedup vs baseline: 2.0253x; 1.7379x over previous
import functools

import jax
import jax.numpy as jnp
from jax import lax
from jax.experimental import pallas as pl
from jax.experimental.pallas import tpu as pltpu
from jax.experimental.pallas import tpu_sc as plsc

NC, NS, LANES = 2, 16, 16
NW = NC * NS
IROW = 128
SUB = 4
CHUNK = IROW * SUB


def _make_transpose_kernel(v: int, d: int):
    nblk = v // 128
    per_w = nblk // NW
    extra = nblk - per_w * NW
    tail_rows = v - nblk * 128
    out_sds = jax.ShapeDtypeStruct((v * d,), jnp.float32)

    @functools.partial(
        pl.kernel,
        out_type=(out_sds, out_sds),
        mesh=plsc.VectorSubcoreMesh(
            core_axis_name="c", subcore_axis_name="s",
            num_cores=NC, num_subcores=NS),
        scratch_types=[
            pltpu.VMEM((d, 128), jnp.float32),
            pltpu.VMEM((d, 128), jnp.float32),
            pltpu.VMEM((d, 128), jnp.float32),
            pltpu.VMEM((d, 128), jnp.float32),
            pltpu.VMEM((d * 128,), jnp.float32),
            pltpu.VMEM((d * 128,), jnp.float32),
            pltpu.VMEM((d * 128,), jnp.float32),
            pltpu.VMEM((d * 128,), jnp.float32),
            pltpu.VMEM((tail_rows * d or 128,), jnp.float32),
            pltpu.SemaphoreType.DMA,
            pltpu.SemaphoreType.DMA,
            pltpu.SemaphoreType.DMA,
            pltpu.SemaphoreType.DMA,
        ],
        compiler_params=pltpu.CompilerParams(
            use_tc_tiling_on_sc=True, needs_layout_passes=False),
    )
    def k(ett_hbm, etail_hbm, gtt_hbm, gtail_hbm, eout_hbm, gout_hbm,
          ein0, ein1, gin0, gin1, eo0, eo1, go0, go1, tbuf,
          gs0, gs1, os0, os1):
        wid = lax.axis_index("s") * NC + lax.axis_index("c")
        base = wid * per_w
        inb = ((ein0, gin0), (ein1, gin1))
        outb = ((eo0, go0), (eo1, go1))
        gsem = (gs0, gs1)
        osem = (os0, os1)
        srcs = (ett_hbm, gtt_hbm)
        dsts = (eout_hbm, gout_hbm)
        owords = d * 128

        def fire_in(blk, slot):
            for t in range(2):
                pltpu.async_copy(
                    srcs[t].at[:, pl.ds(blk * 128, 128)], inb[slot][t],
                    gsem[slot])

        def wait_in(slot):
            for t in range(2):
                pltpu.make_async_copy(
                    srcs[t].at[:, pl.ds(0, 128)], inb[slot][t],
                    gsem[slot]).wait()

        def transpose(slot):
            iot = lax.iota(jnp.int32, LANES)

            def diag(c0, carry):
                cvec = c0 + iot
                cvec = jnp.where(cvec < d, cvec, cvec - d)
                sbase = iot * d + cvec
                for t in range(2):
                    for r0 in range(0, 128, LANES):
                        val = plsc.load_gather(inb[slot][t], [cvec, r0 + iot])
                        plsc.store_scatter(outb[slot][t],
                                           [sbase + r0 * d], val)
                return carry

            lax.fori_loop(0, d, diag, 0, unroll=4)

        def fire_out(blk, slot):
            for t in range(2):
                pltpu.async_copy(
                    outb[slot][t], dsts[t].at[pl.ds(blk * owords, owords)],
                    osem[slot])

        def wait_out(slot):
            for t in range(2):
                pltpu.make_async_copy(
                    outb[slot][t], dsts[t].at[pl.ds(0, owords)],
                    osem[slot]).wait()

        @pl.when(wid < extra)
        def _():
            blk = per_w * NW + wid
            fire_in(blk, 0)
            wait_in(0)
            transpose(0)
            fire_out(blk, 0)
            wait_out(0)

        if tail_rows:
            tw = tail_rows * d
            for t in range(2):
                tails = (etail_hbm, gtail_hbm)

                @pl.when(wid == extra + t)
                def _(t=t):
                    pltpu.sync_copy(tails[t], tbuf)
                    pltpu.sync_copy(tbuf,
                                    dsts[t].at[pl.ds(nblk * owords, tw)])

        fire_in(base, 0)

        def loop_body(j, carry):
            for s in range(2):
                blk = base + j + s
                nxt = j + s + 1
                nslot = 1 - s

                @pl.when(nxt < per_w)
                def _():
                    @pl.when(j + s >= 1)
                    def _():
                        wait_out(nslot)
                    fire_in(base + nxt, nslot)

                wait_in(s)
                transpose(s)
                fire_out(blk, s)
            return carry

        lax.fori_loop(0, per_w // 2, lambda i, c: loop_body(2 * i, c), 0)
        wait_out(0)
        wait_out(1)

    return k


def _make_gather_kernel(n_chunks: int, d: int):
    per_w = n_chunks * CHUNK

    @functools.partial(
        pl.kernel,
        out_type=jax.ShapeDtypeStruct((NW * per_w, d), jnp.float32),
        mesh=plsc.VectorSubcoreMesh(
            core_axis_name="c", subcore_axis_name="s",
            num_cores=NC, num_subcores=NS),
        scratch_types=[
            pltpu.VMEM((n_chunks * SUB, IROW), jnp.int32),
            pltpu.VMEM((CHUNK, d), jnp.float32),
            pltpu.VMEM((CHUNK, d), jnp.float32),
            pltpu.VMEM((CHUNK, d), jnp.float32),
            pltpu.VMEM((CHUNK, d), jnp.float32),
            pltpu.SemaphoreType.DMA,
            pltpu.SemaphoreType.DMA,
            pltpu.SemaphoreType.DMA,
            pltpu.SemaphoreType.DMA,
        ],
        compiler_params=pltpu.CompilerParams(use_tc_tiling_on_sc=False),
    )
    def k(idx_hbm, emb_hbm, gate_hbm, out_hbm,
          idx_v, emb0, emb1, gate0, gate1, gs0, gs1, os0, os1):
        wid = lax.axis_index("s") * NC + lax.axis_index("c")
        base = wid * per_w
        ebuf = (emb0, emb1)
        gbuf = (gate0, gate1)
        gsem = (gs0, gs1)
        osem = (os0, os1)

        pltpu.sync_copy(idx_hbm.at[wid], idx_v)

        def fire_gathers(chunk, slot):
            for q in range(SUB):
                row = idx_v.at[chunk * SUB + q]
                dst = pl.ds(q * IROW, IROW)
                pltpu.async_copy(emb_hbm.at[row], ebuf[slot].at[dst], gsem[slot])
                pltpu.async_copy(gate_hbm.at[row], gbuf[slot].at[dst], gsem[slot])

        def wait_gathers(slot):
            row = idx_v.at[0]
            pltpu.make_async_copy(emb_hbm.at[row], ebuf[slot], gsem[slot]).wait()
            pltpu.make_async_copy(gate_hbm.at[row], gbuf[slot], gsem[slot]).wait()

        def out_slice(chunk):
            return out_hbm.at[pl.ds(base + chunk * CHUNK, CHUNK)]

        def compute_chunk(slot):
            e, g = ebuf[slot], gbuf[slot]

            def row_body(r, carry):
                for h in range(0, d, LANES):
                    sl = (r, pl.ds(h, LANES))
                    e[sl] = jnp.where(g[sl] >= 0.5, e[sl], 0.0)
                return carry

            lax.fori_loop(0, CHUNK, row_body, 0, unroll=8)

        fire_gathers(0, 0)

        def loop_body(j, carry):
            for b in range(2):
                chunk = j + b
                nxt = chunk + 1
                nslot = 1 - b

                @pl.when(nxt < n_chunks)
                def _():
                    @pl.when(chunk >= 1)
                    def _():
                        pltpu.make_async_copy(
                            ebuf[nslot], out_slice(chunk - 1), osem[nslot]
                        ).wait()
                    fire_gathers(nxt, nslot)

                wait_gathers(b)
                compute_chunk(b)
                pltpu.async_copy(ebuf[b], out_slice(chunk), osem[b])
            return carry

        lax.fori_loop(0, n_chunks // 2, lambda i, c: loop_body(2 * i, c), 0)

        pltpu.make_async_copy(ebuf[0], out_slice(n_chunks - 2), osem[0]).wait()
        pltpu.make_async_copy(ebuf[1], out_slice(n_chunks - 1), osem[1]).wait()

    return k


def kernel(indices, emb_table, gate_table):
    b, f = indices.shape
    v, d = emb_table.shape
    n = b * f
    assert n % (NW * CHUNK) == 0 and d % LANES == 0
    assert (v % 128) * d % 128 == 0 and v * d % 128 == 0
    n_chunks = n // (NW * CHUNK)
    nblk = v // 128
    tail_rows = v - nblk * 128

    def tail(t):
        if tail_rows:
            return t[v - tail_rows:, :].reshape(tail_rows * d)
        return jnp.zeros((128,), jnp.float32)

    wide_e, wide_g = _make_transpose_kernel(v, d)(
        emb_table.T, tail(emb_table), gate_table.T, tail(gate_table))
    idx = indices.astype(jnp.int32).reshape(NW, n_chunks * SUB, IROW)
    out = _make_gather_kernel(n_chunks, d)(
        idx, wide_e.reshape(v, d), wide_g.reshape(v, d))
    return out.reshape(b, f, d)


# speedup vs baseline: 3.0012x; 1.4819x over previous
import functools

import jax
import jax.numpy as jnp
from jax import lax
from jax.experimental import pallas as pl
from jax.experimental.pallas import tpu as pltpu
from jax.experimental.pallas import tpu_sc as plsc

NC, NS, LANES = 2, 16, 16
NW = NC * NS
IROW = 128
SUB = 4
CHUNK = IROW * SUB


SLAB = 256


def _make_transpose_kernel(v: int, d: int):
    nblk = v // SLAB
    per_w = nblk // NW
    extra = nblk - per_w * NW
    tail_rows = v - nblk * SLAB
    out_sds = jax.ShapeDtypeStruct((v * d,), jnp.float32)

    @functools.partial(
        pl.kernel,
        out_type=(out_sds, out_sds),
        mesh=plsc.VectorSubcoreMesh(
            core_axis_name="c", subcore_axis_name="s",
            num_cores=NC, num_subcores=NS),
        scratch_types=[
            pltpu.VMEM((d, SLAB), jnp.float32),
            pltpu.VMEM((d, SLAB), jnp.float32),
            pltpu.VMEM((d, SLAB), jnp.float32),
            pltpu.VMEM((d, SLAB), jnp.float32),
            pltpu.VMEM((d * SLAB,), jnp.float32),
            pltpu.VMEM((d * SLAB,), jnp.float32),
            pltpu.VMEM((d * SLAB,), jnp.float32),
            pltpu.VMEM((d * SLAB,), jnp.float32),
            pltpu.VMEM((tail_rows * d or 128,), jnp.float32),
            pltpu.SemaphoreType.DMA,
            pltpu.SemaphoreType.DMA,
            pltpu.SemaphoreType.DMA,
            pltpu.SemaphoreType.DMA,
        ],
        compiler_params=pltpu.CompilerParams(
            use_tc_tiling_on_sc=True, needs_layout_passes=False),
    )
    def k(ett_hbm, etail_hbm, gtt_hbm, gtail_hbm, eout_hbm, gout_hbm,
          ein0, ein1, gin0, gin1, eo0, eo1, go0, go1, tbuf,
          gs0, gs1, os0, os1):
        wid = lax.axis_index("s") * NC + lax.axis_index("c")
        base = wid * per_w
        inb = ((ein0, gin0), (ein1, gin1))
        outb = ((eo0, go0), (eo1, go1))
        gsem = (gs0, gs1)
        osem = (os0, os1)
        srcs = (ett_hbm, gtt_hbm)
        dsts = (eout_hbm, gout_hbm)
        owords = d * SLAB

        def fire_in(blk, slot):
            for t in range(2):
                pltpu.async_copy(
                    srcs[t].at[:, pl.ds(blk * SLAB, SLAB)], inb[slot][t],
                    gsem[slot])

        def wait_in(slot):
            for t in range(2):
                pltpu.make_async_copy(
                    srcs[t].at[:, pl.ds(0, SLAB)], inb[slot][t],
                    gsem[slot]).wait()

        def transpose(slot):
            iot = lax.iota(jnp.int32, LANES)

            def diag(c0, carry):
                cvec = c0 + iot
                cvec = jnp.where(cvec < d, cvec, cvec - d)
                sbase = iot * d + cvec
                for t in range(2):
                    vals = [plsc.load_gather(inb[slot][t], [cvec, r0 + iot])
                            for r0 in range(0, SLAB, LANES)]
                    for i, r0 in enumerate(range(0, SLAB, LANES)):
                        plsc.store_scatter(outb[slot][t],
                                           [sbase + r0 * d], vals[i])
                return carry

            lax.fori_loop(0, d, diag, 0, unroll=2)

        def fire_out(blk, slot):
            for t in range(2):
                pltpu.async_copy(
                    outb[slot][t], dsts[t].at[pl.ds(blk * owords, owords)],
                    osem[slot])

        def wait_out(slot):
            for t in range(2):
                pltpu.make_async_copy(
                    outb[slot][t], dsts[t].at[pl.ds(0, owords)],
                    osem[slot]).wait()

        @pl.when(wid < extra)
        def _():
            blk = per_w * NW + wid
            fire_in(blk, 0)
            wait_in(0)
            transpose(0)
            fire_out(blk, 0)
            wait_out(0)

        if tail_rows:
            tw = tail_rows * d
            for t in range(2):
                tails = (etail_hbm, gtail_hbm)

                @pl.when(wid == extra + t)
                def _(t=t):
                    pltpu.sync_copy(tails[t], tbuf)
                    pltpu.sync_copy(tbuf,
                                    dsts[t].at[pl.ds(nblk * owords, tw)])

        fire_in(base, 0)

        def loop_body(j, carry):
            for s in range(2):
                blk = base + j + s
                nxt = j + s + 1
                nslot = 1 - s

                @pl.when(nxt < per_w)
                def _():
                    @pl.when(j + s >= 1)
                    def _():
                        wait_out(nslot)
                    fire_in(base + nxt, nslot)

                wait_in(s)
                transpose(s)
                fire_out(blk, s)
            return carry

        lax.fori_loop(0, per_w // 2, lambda i, c: loop_body(2 * i, c), 0)
        wait_out(0)
        wait_out(1)

    return k


def _make_gather_kernel(n_chunks: int, d: int):
    per_w = n_chunks * CHUNK

    @functools.partial(
        pl.kernel,
        out_type=jax.ShapeDtypeStruct((NW * per_w, d), jnp.float32),
        mesh=plsc.VectorSubcoreMesh(
            core_axis_name="c", subcore_axis_name="s",
            num_cores=NC, num_subcores=NS),
        scratch_types=[
            pltpu.VMEM((n_chunks * SUB, IROW), jnp.int32),
            pltpu.VMEM((CHUNK, d), jnp.float32),
            pltpu.VMEM((CHUNK, d), jnp.float32),
            pltpu.VMEM((CHUNK, d), jnp.float32),
            pltpu.VMEM((CHUNK, d), jnp.float32),
            pltpu.SemaphoreType.DMA,
            pltpu.SemaphoreType.DMA,
            pltpu.SemaphoreType.DMA,
            pltpu.SemaphoreType.DMA,
        ],
        compiler_params=pltpu.CompilerParams(use_tc_tiling_on_sc=False),
    )
    def k(idx_hbm, emb_hbm, gate_hbm, out_hbm,
          idx_v, emb0, emb1, gate0, gate1, gs0, gs1, os0, os1):
        wid = lax.axis_index("s") * NC + lax.axis_index("c")
        base = wid * per_w
        ebuf = (emb0, emb1)
        gbuf = (gate0, gate1)
        gsem = (gs0, gs1)
        osem = (os0, os1)

        pltpu.sync_copy(idx_hbm.at[wid], idx_v)

        def fire_gathers(chunk, slot):
            for q in range(SUB):
                row = idx_v.at[chunk * SUB + q]
                dst = pl.ds(q * IROW, IROW)
                pltpu.async_copy(emb_hbm.at[row], ebuf[slot].at[dst], gsem[slot])
                pltpu.async_copy(gate_hbm.at[row], gbuf[slot].at[dst], gsem[slot])

        def wait_gathers(slot):
            row = idx_v.at[0]
            pltpu.make_async_copy(emb_hbm.at[row], ebuf[slot], gsem[slot]).wait()
            pltpu.make_async_copy(gate_hbm.at[row], gbuf[slot], gsem[slot]).wait()

        def out_slice(chunk):
            return out_hbm.at[pl.ds(base + chunk * CHUNK, CHUNK)]

        def compute_chunk(slot):
            e, g = ebuf[slot], gbuf[slot]

            def row_body(r, carry):
                for h in range(0, d, LANES):
                    sl = (r, pl.ds(h, LANES))
                    e[sl] = jnp.where(g[sl] >= 0.5, e[sl], 0.0)
                return carry

            lax.fori_loop(0, CHUNK, row_body, 0, unroll=8)

        fire_gathers(0, 0)

        def loop_body(j, carry):
            for b in range(2):
                chunk = j + b
                nxt = chunk + 1
                nslot = 1 - b

                @pl.when(nxt < n_chunks)
                def _():
                    @pl.when(chunk >= 1)
                    def _():
                        pltpu.make_async_copy(
                            ebuf[nslot], out_slice(chunk - 1), osem[nslot]
                        ).wait()
                    fire_gathers(nxt, nslot)

                wait_gathers(b)
                compute_chunk(b)
                pltpu.async_copy(ebuf[b], out_slice(chunk), osem[b])
            return carry

        lax.fori_loop(0, n_chunks // 2, lambda i, c: loop_body(2 * i, c), 0)

        pltpu.make_async_copy(ebuf[0], out_slice(n_chunks - 2), osem[0]).wait()
        pltpu.make_async_copy(ebuf[1], out_slice(n_chunks - 1), osem[1]).wait()

    return k


def kernel(indices, emb_table, gate_table):
    b, f = indices.shape
    v, d = emb_table.shape
    n = b * f
    assert n % (NW * CHUNK) == 0 and d % LANES == 0
    assert (v % 128) * d % 128 == 0 and v * d % 128 == 0
    n_chunks = n // (NW * CHUNK)
    tail_rows = v % SLAB

    def tail(t):
        if tail_rows:
            return t[v - tail_rows:, :].reshape(tail_rows * d)
        return jnp.zeros((128,), jnp.float32)

    wide_e, wide_g = _make_transpose_kernel(v, d)(
        emb_table.T, tail(emb_table), gate_table.T, tail(gate_table))
    idx = indices.astype(jnp.int32).reshape(NW, n_chunks * SUB, IROW)
    out = _make_gather_kernel(n_chunks, d)(
        idx, wide_e.reshape(v, d), wide_g.reshape(v, d))
    return out.reshape(b, f, d)
